# Initial kernel scaffold; baseline (speedup 1.0000x reference)
import functools

import jax
import jax.numpy as jnp
from jax import lax
from jax.experimental import pallas as pl
from jax.experimental.pallas import tpu as pltpu
from jax.experimental.pallas import tpu_sc as plsc

VOCAB = 100000
EMB_DIM = 64

_info = plsc.get_sparse_core_info()
NC, NS = _info.num_cores, _info.num_subcores
NW = NC * NS

CHUNK = 128
NBUF = 4


def _make_kernel(B, D):
    assert B % (NW * CHUNK) == 0
    b_per_w = B // NW
    n_chunks = b_per_w // CHUNK
    n_groups = n_chunks // NBUF
    rows_per_w = n_chunks

    mesh = plsc.VectorSubcoreMesh(core_axis_name="c", subcore_axis_name="s")

    @functools.partial(
        pl.kernel,
        out_type=jax.ShapeDtypeStruct((B, D), jnp.float32),
        mesh=mesh,
        scratch_types=[
            pltpu.VMEM((rows_per_w, CHUNK), jnp.int32),
            pltpu.VMEM((NBUF, CHUNK, D), jnp.float32),
            pltpu.SemaphoreType.DMA,
            pltpu.SemaphoreType.DMA((NBUF,)),
            pltpu.SemaphoreType.DMA((NBUF,)),
        ],
    )
    def emb_kernel(idx_hbm, table_hbm, out_hbm, idx_v, bufs, isem, gsem, wsem):
        wid = lax.axis_index("s") * NC + lax.axis_index("c")
        row_base = wid * rows_per_w
        out_base = wid * b_per_w

        pltpu.make_async_copy(
            idx_hbm.at[pl.ds(row_base, rows_per_w)], idx_v, isem
        ).start()
        pltpu.make_async_copy(
            idx_hbm.at[pl.ds(row_base, rows_per_w)], idx_v, isem
        ).wait()

        def start_gather(c, b):
            pltpu.make_async_copy(
                table_hbm.at[idx_v.at[c]], bufs.at[b], gsem.at[b]
            ).start()

        def wait_gather(b):
            pltpu.make_async_copy(
                table_hbm.at[idx_v.at[0]], bufs.at[b], gsem.at[b]
            ).wait()

        def start_write(c, b):
            pltpu.make_async_copy(
                bufs.at[b], out_hbm.at[pl.ds(out_base + c * CHUNK, CHUNK)],
                wsem.at[b],
            ).start()

        def wait_write(b):
            pltpu.make_async_copy(
                bufs.at[b], out_hbm.at[pl.ds(out_base, CHUNK)], wsem.at[b]
            ).wait()

        for b in range(NBUF):
            start_gather(b, b)

        def body(g, carry):
            for b in range(NBUF):
                wait_gather(b)
                start_write(g * NBUF + b, b)
            for b in range(NBUF):
                wait_write(b)
                start_gather((g + 1) * NBUF + b, b)
            return carry

        lax.fori_loop(0, n_groups - 1, body, 0)

        g_last = n_groups - 1
        for b in range(NBUF):
            wait_gather(b)
            start_write(g_last * NBUF + b, b)
        for b in range(NBUF):
            wait_write(b)

    return emb_kernel


_kernel_fn = _make_kernel(4096 * 200, EMB_DIM)


@jax.jit
def kernel(x, table):
    idx = x.reshape(-1, CHUNK)
    out = _kernel_fn(idx, table)
    return out.reshape(x.shape[0], x.shape[1], EMB_DIM)


# speedup vs baseline: 4.2338x; 4.2338x over previous
import functools

import jax
import jax.numpy as jnp
from jax import lax
from jax.experimental import pallas as pl
from jax.experimental.pallas import tpu as pltpu
from jax.experimental.pallas import tpu_sc as plsc

VOCAB = 100000
EMB_DIM = 64

_info = plsc.get_sparse_core_info()
NC, NS = _info.num_cores, _info.num_subcores
NW = NC * NS

CHUNK = 128
NBUF = 4


def _make_kernel(B, D):
    assert B % (NW * CHUNK) == 0
    b_per_w = B // NW
    n_chunks = b_per_w // CHUNK
    n_groups = n_chunks // NBUF
    rows_per_w = n_chunks

    mesh = plsc.VectorSubcoreMesh(core_axis_name="c", subcore_axis_name="s")

    @functools.partial(
        pl.kernel,
        out_type=jax.ShapeDtypeStruct((B, D), jnp.float32),
        mesh=mesh,
        compiler_params=pltpu.CompilerParams(use_tc_tiling_on_sc=False),
        scratch_types=[
            pltpu.VMEM((rows_per_w, CHUNK), jnp.int32),
            pltpu.VMEM((NBUF, CHUNK, D), jnp.float32),
            pltpu.SemaphoreType.DMA,
            pltpu.SemaphoreType.DMA((NBUF,)),
            pltpu.SemaphoreType.DMA((NBUF,)),
        ],
    )
    def emb_kernel(idx_hbm, table_hbm, out_hbm, idx_v, bufs, isem, gsem, wsem):
        wid = lax.axis_index("s") * NC + lax.axis_index("c")
        row_base = wid * rows_per_w
        out_base = wid * b_per_w

        pltpu.make_async_copy(
            idx_hbm.at[pl.ds(row_base, rows_per_w)], idx_v, isem
        ).start()
        pltpu.make_async_copy(
            idx_hbm.at[pl.ds(row_base, rows_per_w)], idx_v, isem
        ).wait()

        def start_gather(c, b):
            pltpu.make_async_copy(
                table_hbm.at[idx_v.at[c]], bufs.at[b], gsem.at[b]
            ).start()

        def wait_gather(b):
            pltpu.make_async_copy(
                table_hbm.at[idx_v.at[0]], bufs.at[b], gsem.at[b]
            ).wait()

        def start_write(c, b):
            pltpu.make_async_copy(
                bufs.at[b], out_hbm.at[pl.ds(out_base + c * CHUNK, CHUNK)],
                wsem.at[b],
            ).start()

        def wait_write(b):
            pltpu.make_async_copy(
                bufs.at[b], out_hbm.at[pl.ds(out_base, CHUNK)], wsem.at[b]
            ).wait()

        for b in range(NBUF):
            start_gather(b, b)

        def body(g, carry):
            for b in range(NBUF):
                wait_gather(b)
                start_write(g * NBUF + b, b)
            for b in range(NBUF):
                wait_write(b)
                start_gather((g + 1) * NBUF + b, b)
            return carry

        lax.fori_loop(0, n_groups - 1, body, 0)

        g_last = n_groups - 1
        for b in range(NBUF):
            wait_gather(b)
            start_write(g_last * NBUF + b, b)
        for b in range(NBUF):
            wait_write(b)

    return emb_kernel


_kernel_fn = _make_kernel(4096 * 200, EMB_DIM)


@jax.jit
def kernel(x, table):
    idx = x.reshape(-1, CHUNK)
    out = _kernel_fn(idx, table)
    return out.reshape(x.shape[0], x.shape[1], EMB_DIM)
